# Initial kernel scaffold; baseline (speedup 1.0000x reference)
import jax
import jax.numpy as jnp
from jax.experimental import pallas as pl


def kernel(feature_map):
    raise NotImplementedError("write your pallas kernel here")


# speedup vs baseline: 7.1533x; 7.1533x over previous
import jax
import jax.numpy as jnp
from jax.experimental import pallas as pl

_K = 8
_ITERS = 10


def _dists(c, x, xsq):
    cx = jax.lax.dot_general(
        c, x, (((1,), (0,)), ((), ())),
        preferred_element_type=jnp.float32,
        precision=jax.lax.Precision.HIGHEST)
    csq = jnp.sum(c * c, axis=1, keepdims=True)
    return csq - 2.0 * cx + xsq


def _argmin_k(d):
    best = d[0:1, :]
    idx = jnp.zeros_like(best, dtype=jnp.int32)
    for k in range(1, _K):
        row = d[k:k + 1, :]
        m = row < best
        best = jnp.where(m, row, best)
        idx = jnp.where(m, k, idx)
    return idx


def _segment(lab, x):
    iota = jax.lax.broadcasted_iota(jnp.int32, (_K, x.shape[1]), 0)
    oh = (lab == iota).astype(jnp.float32)
    sums = jax.lax.dot_general(
        oh, x, (((1,), (1,)), ((), ())),
        preferred_element_type=jnp.float32,
        precision=jax.lax.Precision.HIGHEST)
    counts = jnp.sum(oh, axis=1, keepdims=True)
    return sums, counts


def _kmeans_kernel(x_ref, labels_ref, clustered_ref):
    x = x_ref[0]
    N = x.shape[1]
    xsq = jnp.sum(x * x, axis=0, keepdims=True)

    sel_i = jax.lax.broadcasted_iota(jnp.int32, (_K, N), 0)
    sel_n = jax.lax.broadcasted_iota(jnp.int32, (_K, N), 1)
    sel = (sel_i == sel_n).astype(jnp.float32)
    c = jax.lax.dot_general(
        sel, x, (((1,), (1,)), ((), ())),
        preferred_element_type=jnp.float32,
        precision=jax.lax.Precision.HIGHEST)

    for _ in range(_ITERS):
        lab = _argmin_k(_dists(c, x, xsq))
        sums, counts = _segment(lab, x)
        c = jnp.where(counts > 0, sums / jnp.maximum(counts, 1.0), c)

    lab = _argmin_k(_dists(c, x, xsq))
    sums, counts = _segment(lab, x)
    clustered_ref[0] = jnp.where(counts > 0, sums / jnp.maximum(counts, 1.0), 0.0)
    labels_ref[0] = lab


def kernel(feature_map):
    B, C, H, W = feature_map.shape
    N = H * W
    x = feature_map.reshape(B, C, N)
    labels3, clustered = pl.pallas_call(
        _kmeans_kernel,
        grid=(B,),
        in_specs=[pl.BlockSpec((1, C, N), lambda b: (b, 0, 0))],
        out_specs=[
            pl.BlockSpec((1, 1, N), lambda b: (b, 0, 0)),
            pl.BlockSpec((1, _K, C), lambda b: (b, 0, 0)),
        ],
        out_shape=[
            jax.ShapeDtypeStruct((B, 1, N), jnp.int32),
            jax.ShapeDtypeStruct((B, _K, C), jnp.float32),
        ],
    )(x)
    return clustered, labels3.reshape(B, N)


# speedup vs baseline: 7.1587x; 1.0008x over previous
import jax
import jax.numpy as jnp
from jax.experimental import pallas as pl

_K = 8
_ITERS = 10


def _dists(c, x):
    cx = jax.lax.dot_general(
        c, x, (((1,), (0,)), ((), ())),
        preferred_element_type=jnp.float32,
        precision=jax.lax.Precision.HIGHEST)
    csq = jnp.sum(c * c, axis=1, keepdims=True)
    return csq - 2.0 * cx


def _argmin_k(d):
    best = d[0:1, :]
    idx = jnp.zeros_like(best, dtype=jnp.int32)
    for k in range(1, _K):
        row = d[k:k + 1, :]
        m = row < best
        best = jnp.where(m, row, best)
        idx = jnp.where(m, k, idx)
    return idx


def _segment(lab, x):
    iota = jax.lax.broadcasted_iota(jnp.int32, (_K, x.shape[1]), 0)
    oh = (lab == iota).astype(jnp.float32)
    sums = jax.lax.dot_general(
        oh, x, (((1,), (1,)), ((), ())),
        preferred_element_type=jnp.float32,
        precision=jax.lax.Precision.HIGHEST)
    counts = jnp.sum(oh, axis=1, keepdims=True)
    return sums, counts


def _kmeans_kernel(x_ref, labels_ref, clustered_ref):
    x = x_ref[0]
    N = x.shape[1]

    sel_i = jax.lax.broadcasted_iota(jnp.int32, (_K, N), 0)
    sel_n = jax.lax.broadcasted_iota(jnp.int32, (_K, N), 1)
    sel = (sel_i == sel_n).astype(jnp.float32)
    c = jax.lax.dot_general(
        sel, x, (((1,), (1,)), ((), ())),
        preferred_element_type=jnp.float32,
        precision=jax.lax.Precision.HIGHEST)

    for _ in range(_ITERS):
        lab = _argmin_k(_dists(c, x))
        sums, counts = _segment(lab, x)
        c = jnp.where(counts > 0, sums / jnp.maximum(counts, 1.0), c)

    lab = _argmin_k(_dists(c, x))
    sums, counts = _segment(lab, x)
    clustered_ref[0] = jnp.where(counts > 0, sums / jnp.maximum(counts, 1.0), 0.0)
    labels_ref[0] = lab


def kernel(feature_map):
    B, C, H, W = feature_map.shape
    N = H * W
    x = feature_map.reshape(B, C, N)
    labels3, clustered = pl.pallas_call(
        _kmeans_kernel,
        grid=(B,),
        in_specs=[pl.BlockSpec((1, C, N), lambda b: (b, 0, 0))],
        out_specs=[
            pl.BlockSpec((1, 1, N), lambda b: (b, 0, 0)),
            pl.BlockSpec((1, _K, C), lambda b: (b, 0, 0)),
        ],
        out_shape=[
            jax.ShapeDtypeStruct((B, 1, N), jnp.int32),
            jax.ShapeDtypeStruct((B, _K, C), jnp.float32),
        ],
    )(x)
    return clustered, labels3.reshape(B, N)


# speedup vs baseline: 13.1537x; 1.8374x over previous
import jax
import jax.numpy as jnp
from jax.experimental import pallas as pl

_K = 8
_ITERS = 10
_B = 4


def _dists(c, x):
    cx = jax.lax.dot_general(
        c, x, (((1,), (0,)), ((), ())),
        preferred_element_type=jnp.float32,
        precision=jax.lax.Precision.HIGHEST)
    csq = jnp.sum(c * c, axis=1, keepdims=True)
    return csq - 2.0 * cx


def _argmin_k(d):
    best = d[0:1, :]
    idx = jnp.zeros_like(best, dtype=jnp.int32)
    for k in range(1, _K):
        row = d[k:k + 1, :]
        m = row < best
        best = jnp.where(m, row, best)
        idx = jnp.where(m, k, idx)
    return idx


def _segment(lab, x):
    iota = jax.lax.broadcasted_iota(jnp.int32, (_K, x.shape[1]), 0)
    oh = (lab == iota).astype(jnp.float32)
    sums = jax.lax.dot_general(
        oh, x, (((1,), (1,)), ((), ())),
        preferred_element_type=jnp.float32,
        precision=jax.lax.Precision.HIGHEST)
    counts = jnp.sum(oh, axis=1, keepdims=True)
    return sums, counts


def _kmeans_kernel(x_ref, labels_ref, clustered_ref):
    N = x_ref.shape[2]
    xs = [x_ref[b] for b in range(_B)]

    sel_i = jax.lax.broadcasted_iota(jnp.int32, (_K, N), 0)
    sel_n = jax.lax.broadcasted_iota(jnp.int32, (_K, N), 1)
    sel = (sel_i == sel_n).astype(jnp.float32)
    cs = [
        jax.lax.dot_general(
            sel, xs[b], (((1,), (1,)), ((), ())),
            preferred_element_type=jnp.float32,
            precision=jax.lax.Precision.HIGHEST)
        for b in range(_B)
    ]

    for _ in range(_ITERS):
        labs = [_argmin_k(_dists(cs[b], xs[b])) for b in range(_B)]
        for b in range(_B):
            sums, counts = _segment(labs[b], xs[b])
            cs[b] = jnp.where(counts > 0, sums / jnp.maximum(counts, 1.0), cs[b])

    for b in range(_B):
        lab = _argmin_k(_dists(cs[b], xs[b]))
        sums, counts = _segment(lab, xs[b])
        clustered_ref[b] = jnp.where(counts > 0, sums / jnp.maximum(counts, 1.0), 0.0)
        labels_ref[b] = lab


def kernel(feature_map):
    B, C, H, W = feature_map.shape
    N = H * W
    x = feature_map.reshape(B, C, N)
    labels3, clustered = pl.pallas_call(
        _kmeans_kernel,
        out_shape=[
            jax.ShapeDtypeStruct((B, 1, N), jnp.int32),
            jax.ShapeDtypeStruct((B, _K, C), jnp.float32),
        ],
    )(x)
    return clustered, labels3.reshape(B, N)


# speedup vs baseline: 18.2434x; 1.3869x over previous
import jax
import jax.numpy as jnp
from jax.experimental import pallas as pl

_K = 8
_ITERS = 10
_B = 4


def _dot(a, b, dims):
    return jax.lax.dot_general(a, b, (dims, ((), ())),
                               preferred_element_type=jnp.float32)


def _split3(a):
    h = a.astype(jnp.bfloat16)
    l = (a - h.astype(jnp.float32)).astype(jnp.bfloat16)
    l2 = (a - h.astype(jnp.float32) - l.astype(jnp.float32)).astype(jnp.bfloat16)
    return h, l, l2


def _dists(c, x):
    cx = jax.lax.dot_general(
        c, x, (((1,), (0,)), ((), ())),
        preferred_element_type=jnp.float32,
        precision=jax.lax.Precision.HIGHEST)
    csq = jnp.sum(c * c, axis=1, keepdims=True)
    return csq - 2.0 * cx


def _argmin_k(d):
    best = d[0:1, :]
    idx = jnp.zeros_like(best, dtype=jnp.int32)
    for k in range(1, _K):
        row = d[k:k + 1, :]
        m = row < best
        best = jnp.where(m, row, best)
        idx = jnp.where(m, k, idx)
    return idx


def _segment(lab, xh, xl, xl2):
    iota = jax.lax.broadcasted_iota(jnp.int32, (_K, lab.shape[1]), 0)
    ohm = lab == iota
    oh = ohm.astype(jnp.bfloat16)
    dims = ((1,), (1,))
    sums = (_dot(oh, xh, dims) + _dot(oh, xl, dims) + _dot(oh, xl2, dims))
    counts = jnp.sum(ohm.astype(jnp.float32), axis=1, keepdims=True)
    return sums, counts


def _kmeans_kernel(x_ref, labels_ref, clustered_ref):
    N = x_ref.shape[2]
    xs = [x_ref[b] for b in range(_B)]
    xsplit = [_split3(x) for x in xs]

    sel_i = jax.lax.broadcasted_iota(jnp.int32, (_K, N), 0)
    sel_n = jax.lax.broadcasted_iota(jnp.int32, (_K, N), 1)
    sel = (sel_i == sel_n).astype(jnp.bfloat16)
    dims = ((1,), (1,))
    cs = [
        _dot(sel, xh, dims) + _dot(sel, xl, dims) + _dot(sel, xl2, dims)
        for (xh, xl, xl2) in xsplit
    ]

    for _ in range(_ITERS):
        labs = [_argmin_k(_dists(cs[b], xs[b])) for b in range(_B)]
        for b in range(_B):
            sums, counts = _segment(labs[b], *xsplit[b])
            cs[b] = jnp.where(counts > 0, sums / jnp.maximum(counts, 1.0), cs[b])

    for b in range(_B):
        lab = _argmin_k(_dists(cs[b], xs[b]))
        sums, counts = _segment(lab, *xsplit[b])
        clustered_ref[b] = jnp.where(counts > 0, sums / jnp.maximum(counts, 1.0), 0.0)
        labels_ref[b] = lab


def kernel(feature_map):
    B, C, H, W = feature_map.shape
    N = H * W
    x = feature_map.reshape(B, C, N)
    labels3, clustered = pl.pallas_call(
        _kmeans_kernel,
        out_shape=[
            jax.ShapeDtypeStruct((B, 1, N), jnp.int32),
            jax.ShapeDtypeStruct((B, _K, C), jnp.float32),
        ],
    )(x)
    return clustered, labels3.reshape(B, N)
